# baseline (speedup 1.0000x reference)
import jax
import jax.numpy as jnp
from jax.experimental import pallas as pl


_BM = 256
_BN = 2048


def _matmul_kernel(x_ref, w_ref, b_ref, o_ref):
    acc = jax.lax.dot_general(
        x_ref[...], w_ref[...],
        dimension_numbers=(((1,), (1,)), ((), ())),
        precision=jax.lax.Precision.DEFAULT,
        preferred_element_type=jnp.float32,
    )
    o_ref[...] = acc + b_ref[...]


def kernel(input, W, bias):
    B, S, K = input.shape
    N = W.shape[0]
    M = B * S
    x = input.reshape(M, K)
    w = W.astype(jnp.bfloat16)
    b = bias.reshape(1, N)

    grid = (N // _BN, M // _BM)

    out = pl.pallas_call(
        _matmul_kernel,
        grid=grid,
        in_specs=[
            pl.BlockSpec((_BM, K), lambda j, i: (i, 0)),
            pl.BlockSpec((_BN, K), lambda j, i: (j, 0)),
            pl.BlockSpec((1, _BN), lambda j, i: (0, j)),
        ],
        out_specs=pl.BlockSpec((_BM, _BN), lambda j, i: (i, j)),
        out_shape=jax.ShapeDtypeStruct((M, N), jnp.float32),
    )(x, w, b)
    return out.reshape(B, S, N)


# speedup vs baseline: 1.0632x; 1.0632x over previous
import jax
import jax.numpy as jnp
from jax.experimental import pallas as pl
from jax.experimental.pallas import tpu as pltpu


_BM = 512
_BN = 1024


def _matmul_kernel(x_ref, w_ref, b_ref, o_ref):
    acc = jax.lax.dot_general(
        x_ref[...], w_ref[...],
        dimension_numbers=(((1,), (1,)), ((), ())),
        precision=jax.lax.Precision.DEFAULT,
        preferred_element_type=jnp.float32,
    )
    o_ref[...] = acc + b_ref[...]


def kernel(input, W, bias):
    B, S, K = input.shape
    N = W.shape[0]
    M = B * S
    x = input.reshape(M, K)
    b = bias.reshape(1, N)

    grid = (N // _BN, M // _BM)

    out = pl.pallas_call(
        _matmul_kernel,
        grid=grid,
        in_specs=[
            pl.BlockSpec((_BM, K), lambda j, i: (i, 0)),
            pl.BlockSpec((_BN, K), lambda j, i: (j, 0)),
            pl.BlockSpec((1, _BN), lambda j, i: (0, j)),
        ],
        out_specs=pl.BlockSpec((_BM, _BN), lambda j, i: (i, j)),
        out_shape=jax.ShapeDtypeStruct((M, N), jnp.float32),
        compiler_params=pltpu.CompilerParams(
            dimension_semantics=("parallel", "parallel"),
        ),
    )(x, W, b)
    return out.reshape(B, S, N)
